# baseline (speedup 1.0000x reference)
import functools

import jax
import jax.numpy as jnp
from jax import lax
from jax.experimental import pallas as pl
from jax.experimental.pallas import tpu as pltpu
from jax.experimental.pallas import tpu_sc as plsc

_K = 32
_B, _T, _D, _V, _N = 64, 4, 1024, 32000, 100000
_Q = _B * _T
_C = 1000
_NCHUNK = _N // _C
_NW = 32
_RPW = _Q // _NW


def _dist_body(qn_ref, q_ref, keys_ref, kn_ref, out_ref):
    q = q_ref[...]
    keys = keys_ref[...]
    kn = kn_ref[0]
    qn = qn_ref[...]
    dot = jax.lax.dot_general(q, keys, (((1,), (1,)), ((), ())),
                              preferred_element_type=jnp.float32)
    out_ref[0] = qn - 2.0 * dot + kn


def _store_dists(qn, q, keys, kn):
    kn3 = kn.reshape(_NCHUNK, 1, _C)
    return pl.pallas_call(
        _dist_body,
        grid=(_NCHUNK,),
        in_specs=[
            pl.BlockSpec((_Q, 1), lambda i: (0, 0)),
            pl.BlockSpec((_Q, _D), lambda i: (0, 0)),
            pl.BlockSpec((_C, _D), lambda i: (i, 0)),
            pl.BlockSpec((1, 1, _C), lambda i: (i, 0, 0)),
        ],
        out_specs=pl.BlockSpec((1, _Q, _C), lambda i: (i, 0, 0)),
        out_shape=jax.ShapeDtypeStruct((_NCHUNK, _Q, _C), jnp.float32),
        compiler_params=pltpu.CompilerParams(
            dimension_semantics=("arbitrary",)),
    )(qn, q, keys, kn3)


def _sc_topk(dists3):
    mesh = plsc.VectorSubcoreMesh(core_axis_name="c", subcore_axis_name="s")

    @functools.partial(
        pl.kernel, mesh=mesh,
        out_type=[jax.ShapeDtypeStruct((_Q * _K,), jnp.float32),
                  jax.ShapeDtypeStruct((_Q * _K,), jnp.int32)],
        scratch_types=[
            pltpu.VMEM((1024,), jnp.float32),
            pltpu.VMEM((1024,), jnp.float32),
            pltpu.VMEM((64,), jnp.float32),
            pltpu.VMEM((64,), jnp.int32),
            pltpu.SemaphoreType.DMA,
            pltpu.SemaphoreType.DMA,
        ],
    )
    def k(d_hbm, bd_hbm, bi_hbm, buf0, buf1, bestd, besti, sem0, sem1):
        cid = lax.axis_index("c")
        sid = lax.axis_index("s")
        w = sid * 2 + cid
        inf16 = jnp.full((16,), jnp.inf, jnp.float32)
        sems = (sem0, sem1)
        bufs = (buf0, buf1)
        lane0 = lax.broadcasted_iota(jnp.int32, (16,), 0) == 0

        lanei = lax.broadcasted_iota(jnp.int32, (16,), 0)
        _gdn = lax.GatherDimensionNumbers(
            offset_dims=(), collapsed_slice_dims=(0,), start_index_map=(0,))

        def permute(v, perm):
            return lax.gather(
                v, perm[:, None], dimension_numbers=_gdn, slice_sizes=(1,),
                mode=lax.GatherScatterMode.PROMISE_IN_BOUNDS)

        def fold_min(v):
            for sft in (8, 4, 2, 1):
                perm = (lanei + sft) % 16
                v = jnp.minimum(v, permute(v, perm))
            return v[0]

        def fold_sum(v):
            for sft in (8, 4, 2, 1):
                perm = (lanei + sft) % 16
                v = v + permute(v, perm)
            return v[0]

        shr = jnp.maximum(lanei - 1, 0)

        def insert(d_, gl):
            b0 = bestd[pl.ds(0, 16)]
            b1 = bestd[pl.ds(16, 16)]
            cnt = (jnp.where(b0 <= d_, 1, 0) + jnp.where(b1 <= d_, 1, 0))
            pos = fold_sum(cnt)
            t1 = bestd[pl.ds(pos + 16, 16)]
            t0 = bestd[pl.ds(pos, 16)]
            bestd[pl.ds(pos + 17, 16)] = t1
            bestd[pl.ds(pos + 1, 16)] = t0
            sh0 = permute(t0, shr)
            bestd[pl.ds(pos, 16)] = jnp.where(lanei == 0, d_, sh0)
            u1 = besti[pl.ds(pos + 16, 16)]
            u0 = besti[pl.ds(pos, 16)]
            besti[pl.ds(pos + 17, 16)] = u1
            besti[pl.ds(pos + 1, 16)] = u0
            si0 = permute(u0, shr)
            besti[pl.ds(pos, 16)] = jnp.where(lanei == 0, gl, si0)

        def do_row(r, carry):
            row = w * _RPW + r
            bestd[pl.ds(0, 16)] = inf16
            bestd[pl.ds(16, 16)] = inf16
            bestd[pl.ds(32, 16)] = inf16
            bestd[pl.ds(48, 16)] = inf16
            pltpu.async_copy(d_hbm.at[pl.ds(row * _C, _C)],
                             buf0.at[pl.ds(0, _C)], sem0)

            def outer(oc, inner_carry):
                for b in (0, 1):
                    ch = oc * 2 + b
                    nxt = ch + 1

                    @pl.when(nxt < _NCHUNK)
                    def _prefetch():
                        pltpu.async_copy(
                            d_hbm.at[pl.ds((nxt * _Q + row) * _C, _C)],
                            bufs[1 - b].at[pl.ds(0, _C)], sems[1 - b])

                    pltpu.make_async_copy(
                        d_hbm.at[pl.ds((ch * _Q + row) * _C, _C)],
                        bufs[b].at[pl.ds(0, _C)], sems[b]).wait()
                    bufs[b][pl.ds(1000, 16)] = inf16
                    bufs[b][pl.ds(1008, 16)] = inf16

                    def grp(g, gc):
                        off = g * 32
                        v0 = bufs[b][pl.ds(off, 16)]
                        v1 = bufs[b][pl.ds(off + 16, 16)]
                        tau = bestd[pl.ds(16, 16)][15]
                        hit = fold_min(jnp.minimum(v0, v1)) < tau

                        @pl.when(hit)
                        def _slow():
                            for t, vv in ((0, v0), (1, v1)):
                                gb = ch * _C + off + t * 16
                                for l in range(16):
                                    dl = vv[l]
                                    tau_l = bestd[pl.ds(16, 16)][15]

                                    @pl.when(dl < tau_l)
                                    def _ins(dl=dl, gl=gb + l):
                                        insert(dl, gl)
                        return gc

                    jax.lax.fori_loop(0, 32, grp, 0)
                return inner_carry

            jax.lax.fori_loop(0, _NCHUNK // 2, outer, 0)
            pltpu.sync_copy(bestd.at[pl.ds(0, _K)],
                            bd_hbm.at[pl.ds(row * _K, _K)])
            pltpu.sync_copy(besti.at[pl.ds(0, _K)],
                            bi_hbm.at[pl.ds(row * _K, _K)])
            return carry

        jax.lax.fori_loop(0, _RPW, do_row, 0)

    bd, bi = k(dists3.reshape(-1))
    return bd.reshape(_Q, _K), bi.reshape(_Q, _K)


def kernel(decoded_last_hidden, decoded_probs, target,
           optor_keys, optor_vals, const_keys, const_vals):
    noise = jax.random.normal(jax.random.key(1), decoded_last_hidden.shape,
                              dtype=decoded_last_hidden.dtype)
    h = decoded_last_hidden + noise * 5.0
    q = h.reshape(-1, h.shape[-1])
    qn = jnp.sum(q * q, axis=-1, keepdims=True)
    okn = jnp.sum(optor_keys * optor_keys, axis=-1)
    ckn = jnp.sum(const_keys * const_keys, axis=-1)

    odists = _store_dists(qn, q, optor_keys, okn)
    cdists = _store_dists(qn, q, const_keys, ckn)
    od, oi = _sc_topk(odists)
    cd, ci = _sc_topk(cdists)

    optor_v = jnp.take(optor_vals, oi, axis=0).reshape(_B, _T, _K)
    const_v = jnp.take(const_vals, ci, axis=0).reshape(_B, _T, _K)
    optor_d = od.reshape(_B, _T, _K)
    const_d = cd.reshape(_B, _T, _K)
    return (decoded_probs, h, target, optor_v, optor_d, const_v, const_d)


# speedup vs baseline: 1.5827x; 1.5827x over previous
import functools

import jax
import jax.numpy as jnp
from jax import lax
from jax.experimental import pallas as pl
from jax.experimental.pallas import tpu as pltpu
from jax.experimental.pallas import tpu_sc as plsc

_K = 32
_B, _T, _D, _V, _N = 64, 4, 1024, 32000, 100000
_Q = _B * _T
_C = 1000
_NCHUNK = _N // _C
_NW = 32
_RPW = _Q // _NW


def _dist_body(qn_ref, q_ref, keys_ref, kn_ref, out_ref):
    q = q_ref[...]
    keys = keys_ref[...]
    kn = kn_ref[0]
    qn = qn_ref[...]
    dot = jax.lax.dot_general(q, keys, (((1,), (1,)), ((), ())),
                              preferred_element_type=jnp.float32)
    out_ref[0] = qn - 2.0 * dot + kn


def _store_dists(qn, q, keys, kn):
    kn3 = kn.reshape(_NCHUNK, 1, _C)
    return pl.pallas_call(
        _dist_body,
        grid=(_NCHUNK,),
        in_specs=[
            pl.BlockSpec((_Q, 1), lambda i: (0, 0)),
            pl.BlockSpec((_Q, _D), lambda i: (0, 0)),
            pl.BlockSpec((_C, _D), lambda i: (i, 0)),
            pl.BlockSpec((1, 1, _C), lambda i: (i, 0, 0)),
        ],
        out_specs=pl.BlockSpec((1, _Q, _C), lambda i: (i, 0, 0)),
        out_shape=jax.ShapeDtypeStruct((_NCHUNK, _Q, _C), jnp.float32),
        compiler_params=pltpu.CompilerParams(
            dimension_semantics=("arbitrary",)),
    )(qn, q, keys, kn3)


def _sc_topk(dists3):
    mesh = plsc.VectorSubcoreMesh(core_axis_name="c", subcore_axis_name="s")

    @functools.partial(
        pl.kernel, mesh=mesh,
        out_type=[jax.ShapeDtypeStruct((_Q * _K,), jnp.float32),
                  jax.ShapeDtypeStruct((_Q * _K,), jnp.int32)],
        scratch_types=[
            pltpu.VMEM((1024,), jnp.float32),
            pltpu.VMEM((1024,), jnp.float32),
            pltpu.VMEM((64,), jnp.float32),
            pltpu.VMEM((64,), jnp.int32),
            pltpu.SemaphoreType.DMA,
            pltpu.SemaphoreType.DMA,
        ],
    )
    def k(d_hbm, bd_hbm, bi_hbm, buf0, buf1, bestd, besti, sem0, sem1):
        cid = lax.axis_index("c")
        sid = lax.axis_index("s")
        w = sid * 2 + cid
        inf16 = jnp.full((16,), jnp.inf, jnp.float32)
        sems = (sem0, sem1)
        bufs = (buf0, buf1)
        lane0 = lax.broadcasted_iota(jnp.int32, (16,), 0) == 0

        lanei = lax.broadcasted_iota(jnp.int32, (16,), 0)
        _gdn = lax.GatherDimensionNumbers(
            offset_dims=(), collapsed_slice_dims=(0,), start_index_map=(0,))

        def permute(v, perm):
            return lax.gather(
                v, perm[:, None], dimension_numbers=_gdn, slice_sizes=(1,),
                mode=lax.GatherScatterMode.PROMISE_IN_BOUNDS)

        def fold_min(v):
            for sft in (8, 4, 2, 1):
                perm = (lanei + sft) % 16
                v = jnp.minimum(v, permute(v, perm))
            return v[0]

        def fold_sum(v):
            for sft in (8, 4, 2, 1):
                perm = (lanei + sft) % 16
                v = v + permute(v, perm)
            return v[0]

        shr = jnp.maximum(lanei - 1, 0)

        def insert(d_, gl):
            b0 = bestd[pl.ds(0, 16)]
            b1 = bestd[pl.ds(16, 16)]
            cnt = (jnp.where(b0 <= d_, 1, 0) + jnp.where(b1 <= d_, 1, 0))
            pos = fold_sum(cnt)
            t1 = bestd[pl.ds(pos + 16, 16)]
            t0 = bestd[pl.ds(pos, 16)]
            bestd[pl.ds(pos + 17, 16)] = t1
            bestd[pl.ds(pos + 1, 16)] = t0
            sh0 = permute(t0, shr)
            bestd[pl.ds(pos, 16)] = jnp.where(lanei == 0, d_, sh0)
            u1 = besti[pl.ds(pos + 16, 16)]
            u0 = besti[pl.ds(pos, 16)]
            besti[pl.ds(pos + 17, 16)] = u1
            besti[pl.ds(pos + 1, 16)] = u0
            si0 = permute(u0, shr)
            besti[pl.ds(pos, 16)] = jnp.where(lanei == 0, gl, si0)

        def do_row(r, carry):
            row = w * _RPW + r
            bestd[pl.ds(0, 16)] = inf16
            bestd[pl.ds(16, 16)] = inf16
            bestd[pl.ds(32, 16)] = inf16
            bestd[pl.ds(48, 16)] = inf16
            pltpu.async_copy(d_hbm.at[pl.ds(row * _C, _C)],
                             buf0.at[pl.ds(0, _C)], sem0)

            def outer(oc, inner_carry):
                for b in (0, 1):
                    ch = oc * 2 + b
                    nxt = ch + 1

                    @pl.when(nxt < _NCHUNK)
                    def _prefetch():
                        pltpu.async_copy(
                            d_hbm.at[pl.ds((nxt * _Q + row) * _C, _C)],
                            bufs[1 - b].at[pl.ds(0, _C)], sems[1 - b])

                    pltpu.make_async_copy(
                        d_hbm.at[pl.ds((ch * _Q + row) * _C, _C)],
                        bufs[b].at[pl.ds(0, _C)], sems[b]).wait()
                    bufs[b][pl.ds(1000, 16)] = inf16
                    bufs[b][pl.ds(1008, 16)] = inf16

                    accs = [bufs[b][pl.ds(k * 16, 16)] for k in range(4)]
                    for j in range(4, 64, 4):
                        for k2 in range(4):
                            accs[k2] = jnp.minimum(
                                accs[k2], bufs[b][pl.ds((j + k2) * 16, 16)])
                    acc = jnp.minimum(jnp.minimum(accs[0], accs[1]),
                                      jnp.minimum(accs[2], accs[3]))
                    tau_c = bestd[pl.ds(16, 16)][15]
                    chunk_hit = fold_min(acc) < tau_c

                    @pl.when(chunk_hit)
                    def _rescan():
                        def blk(bk, bc):
                            boff = bk * 128
                            a = bufs[b][pl.ds(boff, 16)]
                            for j in range(1, 8):
                                a = jnp.minimum(
                                    a, bufs[b][pl.ds(boff + j * 16, 16)])
                            bhit = (fold_min(a) <
                                    bestd[pl.ds(16, 16)][15])

                            @pl.when(bhit)
                            def _scan():
                                def vrg(vi, vc):
                                    voff = boff + vi * 16
                                    v = bufs[b][pl.ds(voff, 16)]
                                    vhit = (fold_min(v) <
                                            bestd[pl.ds(16, 16)][15])

                                    @pl.when(vhit)
                                    def _lanes():
                                        gb = ch * _C + voff
                                        for l in range(16):
                                            dl = v[l]
                                            tau_l = bestd[pl.ds(16, 16)][15]

                                            @pl.when(dl < tau_l)
                                            def _ins(dl=dl, gl=gb + l):
                                                insert(dl, gl)
                                    return vc
                                jax.lax.fori_loop(0, 8, vrg, 0)
                            return bc
                        jax.lax.fori_loop(0, 8, blk, 0)
                return inner_carry

            jax.lax.fori_loop(0, _NCHUNK // 2, outer, 0)
            pltpu.sync_copy(bestd.at[pl.ds(0, _K)],
                            bd_hbm.at[pl.ds(row * _K, _K)])
            pltpu.sync_copy(besti.at[pl.ds(0, _K)],
                            bi_hbm.at[pl.ds(row * _K, _K)])
            return carry

        jax.lax.fori_loop(0, _RPW, do_row, 0)

    bd, bi = k(dists3.reshape(-1))
    return bd.reshape(_Q, _K), bi.reshape(_Q, _K)


def kernel(decoded_last_hidden, decoded_probs, target,
           optor_keys, optor_vals, const_keys, const_vals):
    noise = jax.random.normal(jax.random.key(1), decoded_last_hidden.shape,
                              dtype=decoded_last_hidden.dtype)
    h = decoded_last_hidden + noise * 5.0
    q = h.reshape(-1, h.shape[-1])
    qn = jnp.sum(q * q, axis=-1, keepdims=True)
    okn = jnp.sum(optor_keys * optor_keys, axis=-1)
    ckn = jnp.sum(const_keys * const_keys, axis=-1)

    odists = _store_dists(qn, q, optor_keys, okn)
    cdists = _store_dists(qn, q, const_keys, ckn)
    od, oi = _sc_topk(odists)
    cd, ci = _sc_topk(cdists)

    optor_v = jnp.take(optor_vals, oi, axis=0).reshape(_B, _T, _K)
    const_v = jnp.take(const_vals, ci, axis=0).reshape(_B, _T, _K)
    optor_d = od.reshape(_B, _T, _K)
    const_d = cd.reshape(_B, _T, _K)
    return (decoded_probs, h, target, optor_v, optor_d, const_v, const_d)


# speedup vs baseline: 1.6580x; 1.0476x over previous
import functools

import jax
import jax.numpy as jnp
from jax import lax
from jax.experimental import pallas as pl
from jax.experimental.pallas import tpu as pltpu
from jax.experimental.pallas import tpu_sc as plsc

_K = 32
_B, _T, _D, _V, _N = 64, 4, 1024, 32000, 100000
_Q = _B * _T
_C = 1000
_NCHUNK = _N // _C
_NW = 32
_RPW = _Q // _NW


def _dist_body(qn_ref, q_ref, keys_ref, kn_ref, out_ref):
    q = q_ref[...]
    keys = keys_ref[...]
    kn = kn_ref[0]
    qn = qn_ref[...]
    dot = jax.lax.dot_general(q, keys, (((1,), (1,)), ((), ())),
                              preferred_element_type=jnp.float32)
    out_ref[0] = qn - 2.0 * dot + kn


def _store_dists(qn, q, keys, kn):
    kn3 = kn.reshape(_NCHUNK, 1, _C)
    return pl.pallas_call(
        _dist_body,
        grid=(_NCHUNK,),
        in_specs=[
            pl.BlockSpec((_Q, 1), lambda i: (0, 0)),
            pl.BlockSpec((_Q, _D), lambda i: (0, 0)),
            pl.BlockSpec((_C, _D), lambda i: (i, 0)),
            pl.BlockSpec((1, 1, _C), lambda i: (i, 0, 0)),
        ],
        out_specs=pl.BlockSpec((1, _Q, _C), lambda i: (i, 0, 0)),
        out_shape=jax.ShapeDtypeStruct((_NCHUNK, _Q, _C), jnp.float32),
        compiler_params=pltpu.CompilerParams(
            dimension_semantics=("arbitrary",)),
    )(qn, q, keys, kn3)


def _sc_topk(dists3):
    mesh = plsc.VectorSubcoreMesh(core_axis_name="c", subcore_axis_name="s")

    @functools.partial(
        pl.kernel, mesh=mesh,
        out_type=[jax.ShapeDtypeStruct((_Q * _K,), jnp.float32),
                  jax.ShapeDtypeStruct((_Q * _K,), jnp.int32)],
        scratch_types=[
            pltpu.VMEM((8000,), jnp.float32),
            pltpu.VMEM((8000,), jnp.float32),
            pltpu.VMEM((512,), jnp.float32),
            pltpu.VMEM((512,), jnp.int32),
            pltpu.SemaphoreType.DMA,
            pltpu.SemaphoreType.DMA,
        ],
    )
    def k(d_hbm, bd_hbm, bi_hbm, buf0, buf1, bestd, besti, sem0, sem1):
        cid = lax.axis_index("c")
        sid = lax.axis_index("s")
        w = sid * 2 + cid
        row0 = w * _RPW
        inf16 = jnp.full((16,), jnp.inf, jnp.float32)
        sems = (sem0, sem1)
        bufs = (buf0, buf1)
        lanei = lax.broadcasted_iota(jnp.int32, (16,), 0)
        _gdn = lax.GatherDimensionNumbers(
            offset_dims=(), collapsed_slice_dims=(0,), start_index_map=(0,))

        def permute(v, perm):
            return lax.gather(
                v, perm[:, None], dimension_numbers=_gdn, slice_sizes=(1,),
                mode=lax.GatherScatterMode.PROMISE_IN_BOUNDS)

        def fold_min(v):
            for sft in (8, 4, 2, 1):
                perm = (lanei + sft) % 16
                v = jnp.minimum(v, permute(v, perm))
            return v[0]

        def fold_sum(v):
            for sft in (8, 4, 2, 1):
                perm = (lanei + sft) % 16
                v = v + permute(v, perm)
            return v[0]

        shr = jnp.maximum(lanei - 1, 0)

        for t in range(_RPW * 4):
            bestd[pl.ds(t * 16, 16)] = inf16

        def insert(d_, gl, rb):
            b0 = bestd[pl.ds(rb, 16)]
            b1 = bestd[pl.ds(rb + 16, 16)]
            cnt = jnp.where(b0 <= d_, 1, 0) + jnp.where(b1 <= d_, 1, 0)
            pos = rb + fold_sum(cnt)
            t1 = bestd[pl.ds(pos + 16, 16)]
            t0 = bestd[pl.ds(pos, 16)]
            bestd[pl.ds(pos + 17, 16)] = t1
            bestd[pl.ds(pos + 1, 16)] = t0
            sh0 = permute(t0, shr)
            bestd[pl.ds(pos, 16)] = jnp.where(lanei == 0, d_, sh0)
            u1 = besti[pl.ds(pos + 16, 16)]
            u0 = besti[pl.ds(pos, 16)]
            besti[pl.ds(pos + 17, 16)] = u1
            besti[pl.ds(pos + 1, 16)] = u0
            si0 = permute(u0, shr)
            besti[pl.ds(pos, 16)] = jnp.where(lanei == 0, gl, si0)

        pltpu.async_copy(d_hbm.at[pl.ds(row0 * _C, 8 * _C)],
                         buf0.at[pl.ds(0, 8 * _C)], sem0)

        def outer(oc, outer_c):
            for b in (0, 1):
                ch = oc * 2 + b
                nxt = ch + 1

                @pl.when(nxt < _NCHUNK)
                def _prefetch():
                    pltpu.async_copy(
                        d_hbm.at[pl.ds((nxt * _Q + row0) * _C, 8 * _C)],
                        bufs[1 - b].at[pl.ds(0, 8 * _C)], sems[1 - b])

                pltpu.make_async_copy(
                    d_hbm.at[pl.ds((ch * _Q + row0) * _C, 8 * _C)],
                    bufs[b].at[pl.ds(0, 8 * _C)], sems[b]).wait()

                def row_fn(rr, row_c):
                    base = rr * _C
                    rb = rr * 64
                    tau_c = bestd[pl.ds(rb + 16, 16)][15]

                    def ldv(j):
                        v = bufs[b][pl.ds(base + j * 16, 16)]
                        if j == 62:
                            v = jnp.where(lanei < 8, v, jnp.inf)
                        return v

                    accs = [ldv(0), ldv(1), ldv(2), ldv(3)]
                    for j in range(4, 60, 4):
                        for k2 in range(4):
                            accs[k2] = jnp.minimum(accs[k2], ldv(j + k2))
                    accs[0] = jnp.minimum(accs[0], ldv(60))
                    accs[1] = jnp.minimum(accs[1], ldv(61))
                    accs[2] = jnp.minimum(accs[2], ldv(62))
                    acc = jnp.minimum(jnp.minimum(accs[0], accs[1]),
                                      jnp.minimum(accs[2], accs[3]))
                    hit = fold_min(acc) < tau_c

                    @pl.when(hit)
                    def _rescan():
                        def ldm(cb, vi):
                            v = bufs[b][pl.ds(base + cb + vi * 16, 16)]
                            colv = cb + vi * 16 + lanei
                            return jnp.where(colv < _C, v, jnp.inf)

                        def blk(bk, bc):
                            cb = bk * 144
                            a = ldm(cb, 0)
                            for vi in range(1, 9):
                                a = jnp.minimum(a, ldm(cb, vi))
                            bhit = (fold_min(a) <
                                    bestd[pl.ds(rb + 16, 16)][15])

                            @pl.when(bhit)
                            def _scan():
                                def vrg(vi, vc):
                                    v = ldm(cb, vi)
                                    vhit = (fold_min(v) <
                                            bestd[pl.ds(rb + 16, 16)][15])

                                    @pl.when(vhit)
                                    def _lanes():
                                        for l in range(16):
                                            dl = v[l]
                                            tau_l = bestd[
                                                pl.ds(rb + 16, 16)][15]

                                            @pl.when(dl < tau_l)
                                            def _i(dl=dl, l=l):
                                                gl = ch * _C + cb + vi * 16 + l
                                                insert(dl, gl, rb)
                                    return vc
                                jax.lax.fori_loop(0, 9, vrg, 0)
                            return bc
                        jax.lax.fori_loop(0, 7, blk, 0)
                    return row_c

                jax.lax.fori_loop(0, _RPW, row_fn, 0)
            return outer_c

        jax.lax.fori_loop(0, _NCHUNK // 2, outer, 0)
        for rr in range(_RPW):
            pltpu.sync_copy(bestd.at[pl.ds(rr * 64, _K)],
                            bd_hbm.at[pl.ds((row0 + rr) * _K, _K)])
            pltpu.sync_copy(besti.at[pl.ds(rr * 64, _K)],
                            bi_hbm.at[pl.ds((row0 + rr) * _K, _K)])

    bd, bi = k(dists3.reshape(-1))
    return bd.reshape(_Q, _K), bi.reshape(_Q, _K)


def kernel(decoded_last_hidden, decoded_probs, target,
           optor_keys, optor_vals, const_keys, const_vals):
    noise = jax.random.normal(jax.random.key(1), decoded_last_hidden.shape,
                              dtype=decoded_last_hidden.dtype)
    h = decoded_last_hidden + noise * 5.0
    q = h.reshape(-1, h.shape[-1])
    qn = jnp.sum(q * q, axis=-1, keepdims=True)
    okn = jnp.sum(optor_keys * optor_keys, axis=-1)
    ckn = jnp.sum(const_keys * const_keys, axis=-1)

    odists = _store_dists(qn, q, optor_keys, okn)
    cdists = _store_dists(qn, q, const_keys, ckn)
    od, oi = _sc_topk(odists)
    cd, ci = _sc_topk(cdists)

    optor_v = jnp.take(optor_vals, oi, axis=0).reshape(_B, _T, _K)
    const_v = jnp.take(const_vals, ci, axis=0).reshape(_B, _T, _K)
    optor_d = od.reshape(_B, _T, _K)
    const_d = cd.reshape(_B, _T, _K)
    return (decoded_probs, h, target, optor_v, optor_d, const_v, const_d)


# speedup vs baseline: 4.5309x; 2.7327x over previous
import jax
import jax.numpy as jnp
from jax.experimental import pallas as pl
from jax.experimental.pallas import tpu as pltpu

_K = 32
_B, _T, _D, _V, _N = 64, 4, 1024, 32000, 100000
_Q = _B * _T
_C = 1000
_NCHUNK = _N // _C


def _topk_body(qn_ref, q_ref, keys_ref, kn_ref, bd_ref, bi_ref, cd_ref):
    step = pl.program_id(0)

    q = q_ref[...]
    keys = keys_ref[...]
    kn = kn_ref[0]
    qn = qn_ref[...]
    dot = jax.lax.dot_general(q, keys, (((1,), (1,)), ((), ())),
                              preferred_element_type=jnp.float32)
    dists = qn - 2.0 * dot + kn
    base = step * _C
    colf = jax.lax.broadcasted_iota(jnp.int32, (_Q, _C), 1).astype(jnp.float32)

    @pl.when(step == 0)
    def _first():
        cd_ref[...] = dists
        nd, ni = [], []
        for _ in range(_K):
            cd = cd_ref[...]
            m = jnp.min(cd, axis=1, keepdims=True)
            posf = jnp.min(jnp.where(cd == m, colf, 1e9),
                           axis=1, keepdims=True)
            nd.append(m)
            ni.append(base + posf.astype(jnp.int32))
            cd_ref[...] = jnp.where(colf == posf, jnp.inf, cd)
        bd_ref[...] = jnp.concatenate(nd, axis=1)
        bi_ref[...] = jnp.concatenate(ni, axis=1)

    @pl.when(step > 0)
    def _rest():
        tau = bd_ref[:, _K - 1:_K]
        any_beats = jnp.any(dists < tau)
        kcol = jax.lax.broadcasted_iota(jnp.int32, (_Q, _K), 1)

        @pl.when(any_beats)
        def _merge():
            cd_ref[...] = dists

            def body(_):
                cd = cd_ref[...]
                bd = bd_ref[...]
                bi = bi_ref[...]
                m = jnp.min(cd, axis=1, keepdims=True)
                posf = jnp.min(jnp.where(cd == m, colf, 1e9),
                               axis=1, keepdims=True)
                vi = base + posf.astype(jnp.int32)
                cd_ref[...] = jnp.where(colf == posf, jnp.inf, cd)
                ins = m < bd[:, _K - 1:_K]
                posb = jnp.sum((bd <= m).astype(jnp.int32),
                               axis=1, keepdims=True)
                shd = jnp.roll(bd, 1, axis=1)
                shi = jnp.roll(bi, 1, axis=1)
                nbd = jnp.where(kcol < posb, bd,
                                jnp.where(kcol == posb, m, shd))
                nbi = jnp.where(kcol < posb, bi,
                                jnp.where(kcol == posb, vi, shi))
                bd_ref[...] = jnp.where(ins, nbd, bd)
                bi_ref[...] = jnp.where(ins, nbi, bi)
                return jnp.any(ins)

            jax.lax.while_loop(lambda go: go, body, any_beats)


def _store_topk(qn, q, keys, kn, interpret=False):
    kn3 = kn.reshape(_NCHUNK, 1, _C)
    bd, bi = pl.pallas_call(
        _topk_body,
        grid=(_NCHUNK,),
        in_specs=[
            pl.BlockSpec((_Q, 1), lambda i: (0, 0)),
            pl.BlockSpec((_Q, _D), lambda i: (0, 0)),
            pl.BlockSpec((_C, _D), lambda i: (i, 0)),
            pl.BlockSpec((1, 1, _C), lambda i: (i, 0, 0)),
        ],
        out_specs=[
            pl.BlockSpec((_Q, _K), lambda i: (0, 0)),
            pl.BlockSpec((_Q, _K), lambda i: (0, 0)),
        ],
        out_shape=[
            jax.ShapeDtypeStruct((_Q, _K), jnp.float32),
            jax.ShapeDtypeStruct((_Q, _K), jnp.int32),
        ],
        compiler_params=pltpu.CompilerParams(
            dimension_semantics=("arbitrary",)),
        scratch_shapes=[pltpu.VMEM((_Q, _C), jnp.float32)],
        interpret=interpret,
    )(qn, q, keys, kn3)
    return bd, bi


def kernel(decoded_last_hidden, decoded_probs, target,
           optor_keys, optor_vals, const_keys, const_vals):
    noise = jax.random.normal(jax.random.key(1), decoded_last_hidden.shape,
                              dtype=decoded_last_hidden.dtype)
    h = decoded_last_hidden + noise * 5.0
    q = h.reshape(-1, h.shape[-1])
    qn = jnp.sum(q * q, axis=-1, keepdims=True)
    okn = jnp.sum(optor_keys * optor_keys, axis=-1)
    ckn = jnp.sum(const_keys * const_keys, axis=-1)

    od, oi = _store_topk(qn, q, optor_keys, okn)
    cd, ci = _store_topk(qn, q, const_keys, ckn)

    optor_v = jnp.take(optor_vals, oi, axis=0).reshape(_B, _T, _K)
    const_v = jnp.take(const_vals, ci, axis=0).reshape(_B, _T, _K)
    optor_d = od.reshape(_B, _T, _K)
    const_d = cd.reshape(_B, _T, _K)
    return (decoded_probs, h, target, optor_v, optor_d, const_v, const_d)
